# baseline (speedup 1.0000x reference)
import functools

import jax
import jax.numpy as jnp
from jax import lax
from jax.experimental import pallas as pl
from jax.experimental.pallas import tpu as pltpu
from jax.experimental.pallas import tpu_sc as plsc

N = 10000
E = 320000
D_IN = 128
D_OUT = 64
D_HID = 2 * D_OUT

NC = 2
NS = 16
NW = NC * NS
EW = E // NW
CH = 80
NB = 128
EWP = NB * CH
BLK = 32
NBLK = NB // BLK
PCH = 80
PNB = EW // PCH
PK = 3
PGRP = PNB // PK - 1
NP = 10112
RPT = NP // NS

ROWS_B = 10
RB = N // ROWS_B


_MESH = plsc.VectorSubcoreMesh(core_axis_name="c", subcore_axis_name="s")


def _make_sc_deg():
    @functools.partial(
        pl.kernel,
        out_type=[jax.ShapeDtypeStruct((NP, 16), jnp.float32),
                  jax.ShapeDtypeStruct((NP, 16), jnp.float32)],
        mesh=_MESH,
        scratch_types=[
            [pltpu.VMEM((PCH,), jnp.int32) for _ in range(PK)],
            pltpu.VMEM((PCH, 16), jnp.float32),
            pltpu.VMEM_SHARED((NP, 16), jnp.float32),
            pltpu.SemaphoreType.DMA,
            pltpu.SemaphoreType.DMA,
        ],
    )
    def sc_deg(dst_hbm, t_hbm, outA, outB, didx, rows0, acc, sems, semi):
        c = lax.axis_index("c")
        s = lax.axis_index("s")
        wid = s * NC + c
        base = wid * EW
        pltpu.sync_copy(t_hbm.at[pl.ds(s * RPT, RPT)], acc.at[pl.ds(s * RPT, RPT)])

        def wait_s():
            pltpu.make_async_copy(rows0, acc.at[didx[0]], sems).wait()

        def wait_i():
            pltpu.make_async_copy(dst_hbm.at[pl.ds(0, PCH)], didx[0],
                                  semi).wait()

        def prefetch_idx(i, p):
            pltpu.async_copy(dst_hbm.at[pl.ds(base + i * PCH, PCH)],
                             didx[p], semi)

        pltpu.sync_copy(dst_hbm.at[pl.ds(base, PCH)], didx[0])

        def fill(j, carry):
            rows0[j, :] = jnp.full((16,), 1.0, jnp.float32)
            return carry
        lax.fori_loop(0, PCH, fill, 0)
        plsc.subcore_barrier()

        def group(g, carry):
            for p in range(PK):
                i = PK * g + p

                @pl.when(i >= PK - 1)
                def _():
                    wait_s()
                prefetch_idx(i + 1, (p + 1) % PK)

                @pl.when(i >= 1)
                def _():
                    wait_i()
                pltpu.async_copy(rows0, acc.at[didx[p]], sems, add=True)
            return carry

        lax.fori_loop(0, PGRP + 1, group, 0)
        done = PK * (PGRP + 1)
        for e in range(PNB - done):
            j = done + e
            p = j % PK
            wait_s()
            if j + 1 < PNB:
                prefetch_idx(j + 1, (p + 1) % PK)
            wait_i()
            pltpu.async_copy(rows0, acc.at[didx[p]], sems, add=True)
        for _ in range(PK - 1):
            wait_s()

        plsc.subcore_barrier()

        @pl.when(c == 0)
        def _():
            pltpu.sync_copy(acc.at[pl.ds(s * RPT, RPT)],
                            outA.at[pl.ds(s * RPT, RPT)])

        @pl.when(c == 1)
        def _():
            pltpu.sync_copy(acc.at[pl.ds(s * RPT, RPT)],
                            outB.at[pl.ds(s * RPT, RPT)])

    return sc_deg


def _make_sc_prop(D):
    @functools.partial(
        pl.kernel,
        out_type=[jax.ShapeDtypeStruct((NP, D), jnp.float32),
                  jax.ShapeDtypeStruct((NP, D), jnp.float32)],
        mesh=_MESH,
        scratch_types=[
            [pltpu.VMEM((PCH,), jnp.int32) for _ in range(PK)],
            [pltpu.VMEM((PCH,), jnp.int32) for _ in range(PK)],
            [pltpu.VMEM((PCH, D), jnp.float32) for _ in range(PK)],
            pltpu.VMEM_SHARED((NP, D), jnp.float32),
            pltpu.SemaphoreType.DMA,
            pltpu.SemaphoreType.DMA,
            pltpu.SemaphoreType.DMA,
        ],
    )
    def sc_prop(src_hbm, dst_hbm, t_hbm, outA, outB,
                sidx, didx, rows, acc, semg, sems, semi):
        c = lax.axis_index("c")
        s = lax.axis_index("s")
        wid = s * NC + c
        base = wid * EW
        pltpu.sync_copy(t_hbm.at[pl.ds(s * RPT, RPT)], acc.at[pl.ds(s * RPT, RPT)])
        plsc.subcore_barrier()

        def wait_g():
            pltpu.make_async_copy(t_hbm.at[sidx[0]], rows[0], semg).wait()

        def wait_s():
            pltpu.make_async_copy(rows[0], acc.at[didx[0]], sems).wait()

        def wait_i():
            pltpu.make_async_copy(src_hbm.at[pl.ds(0, PCH)], sidx[0],
                                  semi).wait()
            pltpu.make_async_copy(dst_hbm.at[pl.ds(0, PCH)], didx[0],
                                  semi).wait()

        def prefetch_idx(i, p):
            off = base + i * PCH
            pltpu.async_copy(src_hbm.at[pl.ds(off, PCH)], sidx[p], semi)
            pltpu.async_copy(dst_hbm.at[pl.ds(off, PCH)], didx[p], semi)

        def gather(i, p):
            pltpu.async_copy(t_hbm.at[sidx[p]], rows[p], semg)

        def scatter(p):
            pltpu.async_copy(rows[p], acc.at[didx[p]], sems, add=True)

        pltpu.sync_copy(src_hbm.at[pl.ds(base, PCH)], sidx[0])
        pltpu.sync_copy(dst_hbm.at[pl.ds(base, PCH)], didx[0])

        def group(g, carry):
            for p in range(PK):
                i = PK * g + p

                @pl.when(i >= PK - 1)
                def _():
                    wait_s()
                prefetch_idx(i + 1, (p + 1) % PK)

                @pl.when(i >= 1)
                def _():
                    wait_i()
                gather(i, p)
                if p == 0:
                    @pl.when(g >= 1)
                    def _():
                        wait_g()
                        scatter(PK - 1)
                else:
                    wait_g()
                    scatter(p - 1)
            return carry

        lax.fori_loop(0, PGRP + 1, group, 0)
        done = PK * (PGRP + 1)
        for e in range(PNB - done):
            j = done + e
            p = j % PK
            wait_s()
            if j + 1 < PNB:
                prefetch_idx(j + 1, (p + 1) % PK)
            wait_i()
            gather(j, p)
            wait_g()
            scatter((j - 1) % PK)
        wait_g()
        scatter((PNB - 1) % PK)
        for _ in range(PK - 1):
            wait_s()

        plsc.subcore_barrier()

        @pl.when(c == 0)
        def _():
            pltpu.sync_copy(acc.at[pl.ds(s * RPT, RPT)],
                            outA.at[pl.ds(s * RPT, RPT)])

        @pl.when(c == 1)
        def _():
            pltpu.sync_copy(acc.at[pl.ds(s * RPT, RPT)],
                            outB.at[pl.ds(s * RPT, RPT)])

    return sc_prop


_sc_deg = _make_sc_deg()
_sc_prop = _make_sc_prop(D_IN)


def _tc_stage1(x, W1, dp0, dp1):
    def mm_body(x_ref, w_ref, m_ref):
        m_ref[...] = jnp.dot(x_ref[...], w_ref[...],
                             preferred_element_type=jnp.float32)

    m1 = pl.pallas_call(
        mm_body,
        grid=(ROWS_B,),
        in_specs=[
            pl.BlockSpec((RB, D_IN), lambda i: (i, 0)),
            pl.BlockSpec((D_IN, D_HID), lambda i: (0, 0)),
        ],
        out_specs=pl.BlockSpec((RB, D_HID), lambda i: (i, 0)),
        out_shape=jax.ShapeDtypeStruct((NP, D_HID), jnp.float32),
    )(x, W1)

    def body(m_ref, d0_ref, d1_ref, t1_ref, dis_ref):
        deg = d0_ref[...][:, :1] + d1_ref[...][:, :1] - 1.0
        dis = lax.rsqrt(deg)
        t1_ref[...] = m_ref[...] * dis
        dis_ref[...] = dis

    return pl.pallas_call(
        body,
        grid=(ROWS_B,),
        in_specs=[
            pl.BlockSpec((RB, D_HID), lambda i: (i, 0)),
            pl.BlockSpec((RB, 16), lambda i: (i, 0)),
            pl.BlockSpec((RB, 16), lambda i: (i, 0)),
        ],
        out_specs=[
            pl.BlockSpec((RB, D_HID), lambda i: (i, 0)),
            pl.BlockSpec((RB, 1), lambda i: (i, 0)),
        ],
        out_shape=[
            jax.ShapeDtypeStruct((NP, D_HID), jnp.float32),
            jax.ShapeDtypeStruct((N, 1), jnp.float32),
        ],
    )(m1, dp0, dp1)


def _tc_stage2(pa, pb, t1, dis, b1, Wc):
    def body(pa_ref, pb_ref, t1_ref, dis_ref, b_ref, w_ref, t2_ref):
        s = pa_ref[...] + pb_ref[...] - t1_ref[...]
        h = jnp.maximum(s * dis_ref[...] + b_ref[...], 0.0)
        m = jnp.dot(h, w_ref[...], preferred_element_type=jnp.float32)
        t2_ref[...] = m * dis_ref[...]

    return pl.pallas_call(
        body,
        grid=(ROWS_B,),
        in_specs=[
            pl.BlockSpec((RB, D_HID), lambda i: (i, 0)),
            pl.BlockSpec((RB, D_HID), lambda i: (i, 0)),
            pl.BlockSpec((RB, D_HID), lambda i: (i, 0)),
            pl.BlockSpec((RB, 1), lambda i: (i, 0)),
            pl.BlockSpec((1, D_HID), lambda i: (0, 0)),
            pl.BlockSpec((D_HID, 2 * D_OUT), lambda i: (0, 0)),
        ],
        out_specs=pl.BlockSpec((RB, 2 * D_OUT), lambda i: (i, 0)),
        out_shape=jax.ShapeDtypeStruct((NP, 2 * D_OUT), jnp.float32),
    )(pa, pb, t1, dis, b1, Wc)


def _tc_stage3(pa, pb, t2, dis, bmu, bls):
    def body(pa_ref, pb_ref, t2_ref, dis_ref, bm_ref, bl_ref, mu_ref, ls_ref):
        p = (pa_ref[...] + pb_ref[...] - t2_ref[...]) * dis_ref[...]
        mu_ref[...] = p[:, :D_OUT] + bm_ref[...]
        ls_ref[...] = p[:, D_OUT:] + bl_ref[...]

    return pl.pallas_call(
        body,
        grid=(ROWS_B,),
        in_specs=[
            pl.BlockSpec((RB, 2 * D_OUT), lambda i: (i, 0)),
            pl.BlockSpec((RB, 2 * D_OUT), lambda i: (i, 0)),
            pl.BlockSpec((RB, 2 * D_OUT), lambda i: (i, 0)),
            pl.BlockSpec((RB, 1), lambda i: (i, 0)),
            pl.BlockSpec((1, D_OUT), lambda i: (0, 0)),
            pl.BlockSpec((1, D_OUT), lambda i: (0, 0)),
        ],
        out_specs=[
            pl.BlockSpec((RB, D_OUT), lambda i: (i, 0)),
            pl.BlockSpec((RB, D_OUT), lambda i: (i, 0)),
        ],
        out_shape=[
            jax.ShapeDtypeStruct((N, D_OUT), jnp.float32),
            jax.ShapeDtypeStruct((N, D_OUT), jnp.float32),
        ],
    )(pa, pb, t2, dis, bmu, bls)


def kernel(x, edge_index, W1, b1, Wmu, bmu, Wls, bls):
    src = edge_index[0]
    dst = edge_index[1]
    ones16 = jnp.ones((NP, 16), jnp.float32)
    Wc = jnp.concatenate([Wmu, Wls], axis=1)

    dpA, dpB = _sc_deg(dst, ones16)

    t1, dis = _tc_stage1(x, W1, dpA, dpB)

    s1a, s1b = _sc_prop(src, dst, t1)
    t2 = _tc_stage2(s1a, s1b, t1, dis, b1.reshape(1, -1), Wc)

    s2a, s2b = _sc_prop(src, dst, t2)
    mu, ls = _tc_stage3(s2a, s2b, t2, dis,
                        bmu.reshape(1, -1), bls.reshape(1, -1))
    return (mu, ls)


# speedup vs baseline: 1.0573x; 1.0573x over previous
import functools

import jax
import jax.numpy as jnp
from jax import lax
from jax.experimental import pallas as pl
from jax.experimental.pallas import tpu as pltpu
from jax.experimental.pallas import tpu_sc as plsc

N = 10000
E = 320000
D_IN = 128
D_OUT = 64
D_HID = 2 * D_OUT

NC = 2
NS = 16
NW = NC * NS
EW = E // NW
CH = 80
NB = 128
EWP = NB * CH
BLK = 32
NBLK = NB // BLK
PCH = 80
PNB = EW // PCH
PK = 3
PGRP = PNB // PK - 1
NP = 10112
RPT = NP // NS

ROWS_B = 10
RB = N // ROWS_B


_MESH = plsc.VectorSubcoreMesh(core_axis_name="c", subcore_axis_name="s")


def _make_sc_deg():
    @functools.partial(
        pl.kernel,
        out_type=[jax.ShapeDtypeStruct((NP, 16), jnp.float32),
                  jax.ShapeDtypeStruct((NP, 16), jnp.float32)],
        mesh=_MESH,
        scratch_types=[
            pltpu.VMEM((BLK, CH), jnp.int32),
            pltpu.VMEM((BLK, CH), jnp.int32),
            pltpu.VMEM((CH, 16), jnp.float32),
            pltpu.VMEM_SHARED((NP, 16), jnp.float32),
            pltpu.SemaphoreType.DMA,
            pltpu.SemaphoreType.DMA,
        ],
    )
    def sc_deg(dst_hbm, t_hbm, outA, outB, didxA, didxB, rows0, acc, sems, semi):
        c = lax.axis_index("c")
        s = lax.axis_index("s")
        wid = s * NC + c
        dbufs = [didxA, didxB]
        pltpu.sync_copy(t_hbm.at[pl.ds(s * RPT, RPT)], acc.at[pl.ds(s * RPT, RPT)])

        def wait_s():
            pltpu.make_async_copy(rows0, acc.at[didxA.at[0]], sems).wait()

        def load_idx_block(blk, k):
            pltpu.async_copy(dst_hbm.at[wid, pl.ds(blk * BLK, BLK)],
                             dbufs[k], semi)

        def wait_idx_block(k):
            pltpu.make_async_copy(dst_hbm.at[wid, pl.ds(0, BLK)],
                                  dbufs[k], semi).wait()

        load_idx_block(0, 0)

        def fill(j, carry):
            rows0[j, :] = jnp.full((16,), 1.0, jnp.float32)
            return carry
        lax.fori_loop(0, CH, fill, 0)
        plsc.subcore_barrier()
        wait_idx_block(0)

        for blk in range(NBLK):
            k = blk % 2
            didx = dbufs[k]
            if blk + 1 < NBLK:
                load_idx_block(blk + 1, 1 - k)

            def body(i, carry, didx=didx):
                pltpu.async_copy(rows0, acc.at[didx.at[i]], sems, add=True)

                @pl.when(i >= 2)
                def _():
                    wait_s()
                return carry

            lax.fori_loop(0, BLK, body, 0)
            wait_s()
            wait_s()
            if blk + 1 < NBLK:
                wait_idx_block(1 - k)

        plsc.subcore_barrier()

        @pl.when(c == 0)
        def _():
            pltpu.sync_copy(acc.at[pl.ds(s * RPT, RPT)],
                            outA.at[pl.ds(s * RPT, RPT)])

        @pl.when(c == 1)
        def _():
            pltpu.sync_copy(acc.at[pl.ds(s * RPT, RPT)],
                            outB.at[pl.ds(s * RPT, RPT)])

    return sc_deg


def _make_sc_prop(D):
    @functools.partial(
        pl.kernel,
        out_type=[jax.ShapeDtypeStruct((NP, D), jnp.float32),
                  jax.ShapeDtypeStruct((NP, D), jnp.float32)],
        mesh=_MESH,
        scratch_types=[
            [pltpu.VMEM((PCH,), jnp.int32) for _ in range(PK)],
            [pltpu.VMEM((PCH,), jnp.int32) for _ in range(PK)],
            [pltpu.VMEM((PCH, D), jnp.float32) for _ in range(PK)],
            pltpu.VMEM_SHARED((NP, D), jnp.float32),
            pltpu.SemaphoreType.DMA,
            pltpu.SemaphoreType.DMA,
            pltpu.SemaphoreType.DMA,
        ],
    )
    def sc_prop(src_hbm, dst_hbm, t_hbm, outA, outB,
                sidx, didx, rows, acc, semg, sems, semi):
        c = lax.axis_index("c")
        s = lax.axis_index("s")
        wid = s * NC + c
        base = wid * EW
        pltpu.sync_copy(t_hbm.at[pl.ds(s * RPT, RPT)], acc.at[pl.ds(s * RPT, RPT)])
        plsc.subcore_barrier()

        def wait_g():
            pltpu.make_async_copy(t_hbm.at[sidx[0]], rows[0], semg).wait()

        def wait_s():
            pltpu.make_async_copy(rows[0], acc.at[didx[0]], sems).wait()

        def wait_i():
            pltpu.make_async_copy(src_hbm.at[pl.ds(0, PCH)], sidx[0],
                                  semi).wait()
            pltpu.make_async_copy(dst_hbm.at[pl.ds(0, PCH)], didx[0],
                                  semi).wait()

        def prefetch_idx(i, p):
            off = base + i * PCH
            pltpu.async_copy(src_hbm.at[pl.ds(off, PCH)], sidx[p], semi)
            pltpu.async_copy(dst_hbm.at[pl.ds(off, PCH)], didx[p], semi)

        def gather(i, p):
            pltpu.async_copy(t_hbm.at[sidx[p]], rows[p], semg)

        def scatter(p):
            pltpu.async_copy(rows[p], acc.at[didx[p]], sems, add=True)

        pltpu.sync_copy(src_hbm.at[pl.ds(base, PCH)], sidx[0])
        pltpu.sync_copy(dst_hbm.at[pl.ds(base, PCH)], didx[0])

        def group(g, carry):
            for p in range(PK):
                i = PK * g + p

                @pl.when(i >= PK - 1)
                def _():
                    wait_s()
                prefetch_idx(i + 1, (p + 1) % PK)

                @pl.when(i >= 1)
                def _():
                    wait_i()
                gather(i, p)
                if p == 0:
                    @pl.when(g >= 1)
                    def _():
                        wait_g()
                        scatter(PK - 1)
                else:
                    wait_g()
                    scatter(p - 1)
            return carry

        lax.fori_loop(0, PGRP + 1, group, 0)
        done = PK * (PGRP + 1)
        for e in range(PNB - done):
            j = done + e
            p = j % PK
            wait_s()
            if j + 1 < PNB:
                prefetch_idx(j + 1, (p + 1) % PK)
            wait_i()
            gather(j, p)
            wait_g()
            scatter((j - 1) % PK)
        wait_g()
        scatter((PNB - 1) % PK)
        for _ in range(PK - 1):
            wait_s()

        plsc.subcore_barrier()

        @pl.when(c == 0)
        def _():
            pltpu.sync_copy(acc.at[pl.ds(s * RPT, RPT)],
                            outA.at[pl.ds(s * RPT, RPT)])

        @pl.when(c == 1)
        def _():
            pltpu.sync_copy(acc.at[pl.ds(s * RPT, RPT)],
                            outB.at[pl.ds(s * RPT, RPT)])

    return sc_prop


_sc_deg = _make_sc_deg()
_sc_prop = _make_sc_prop(D_IN)


def _tc_stage1(x, W1, dp0, dp1):
    def mm_body(x_ref, w_ref, m_ref):
        m_ref[...] = jnp.dot(x_ref[...], w_ref[...],
                             preferred_element_type=jnp.float32)

    m1 = pl.pallas_call(
        mm_body,
        grid=(ROWS_B,),
        in_specs=[
            pl.BlockSpec((RB, D_IN), lambda i: (i, 0)),
            pl.BlockSpec((D_IN, D_HID), lambda i: (0, 0)),
        ],
        out_specs=pl.BlockSpec((RB, D_HID), lambda i: (i, 0)),
        out_shape=jax.ShapeDtypeStruct((NP, D_HID), jnp.float32),
    )(x, W1)

    def body(m_ref, d0_ref, d1_ref, t1_ref, dis_ref):
        deg = d0_ref[...][:, :1] + d1_ref[...][:, :1] - 1.0
        dis = lax.rsqrt(deg)
        t1_ref[...] = m_ref[...] * dis
        dis_ref[...] = dis

    return pl.pallas_call(
        body,
        grid=(ROWS_B,),
        in_specs=[
            pl.BlockSpec((RB, D_HID), lambda i: (i, 0)),
            pl.BlockSpec((RB, 16), lambda i: (i, 0)),
            pl.BlockSpec((RB, 16), lambda i: (i, 0)),
        ],
        out_specs=[
            pl.BlockSpec((RB, D_HID), lambda i: (i, 0)),
            pl.BlockSpec((RB, 1), lambda i: (i, 0)),
        ],
        out_shape=[
            jax.ShapeDtypeStruct((NP, D_HID), jnp.float32),
            jax.ShapeDtypeStruct((N, 1), jnp.float32),
        ],
    )(m1, dp0, dp1)


def _tc_stage2(pa, pb, t1, dis, b1, Wc):
    def body(pa_ref, pb_ref, t1_ref, dis_ref, b_ref, w_ref, t2_ref):
        s = pa_ref[...] + pb_ref[...] - t1_ref[...]
        h = jnp.maximum(s * dis_ref[...] + b_ref[...], 0.0)
        m = jnp.dot(h, w_ref[...], preferred_element_type=jnp.float32)
        t2_ref[...] = m * dis_ref[...]

    return pl.pallas_call(
        body,
        grid=(ROWS_B,),
        in_specs=[
            pl.BlockSpec((RB, D_HID), lambda i: (i, 0)),
            pl.BlockSpec((RB, D_HID), lambda i: (i, 0)),
            pl.BlockSpec((RB, D_HID), lambda i: (i, 0)),
            pl.BlockSpec((RB, 1), lambda i: (i, 0)),
            pl.BlockSpec((1, D_HID), lambda i: (0, 0)),
            pl.BlockSpec((D_HID, 2 * D_OUT), lambda i: (0, 0)),
        ],
        out_specs=pl.BlockSpec((RB, 2 * D_OUT), lambda i: (i, 0)),
        out_shape=jax.ShapeDtypeStruct((NP, 2 * D_OUT), jnp.float32),
    )(pa, pb, t1, dis, b1, Wc)


def _tc_stage3(pa, pb, t2, dis, bmu, bls):
    def body(pa_ref, pb_ref, t2_ref, dis_ref, bm_ref, bl_ref, mu_ref, ls_ref):
        p = (pa_ref[...] + pb_ref[...] - t2_ref[...]) * dis_ref[...]
        mu_ref[...] = p[:, :D_OUT] + bm_ref[...]
        ls_ref[...] = p[:, D_OUT:] + bl_ref[...]

    return pl.pallas_call(
        body,
        grid=(ROWS_B,),
        in_specs=[
            pl.BlockSpec((RB, 2 * D_OUT), lambda i: (i, 0)),
            pl.BlockSpec((RB, 2 * D_OUT), lambda i: (i, 0)),
            pl.BlockSpec((RB, 2 * D_OUT), lambda i: (i, 0)),
            pl.BlockSpec((RB, 1), lambda i: (i, 0)),
            pl.BlockSpec((1, D_OUT), lambda i: (0, 0)),
            pl.BlockSpec((1, D_OUT), lambda i: (0, 0)),
        ],
        out_specs=[
            pl.BlockSpec((RB, D_OUT), lambda i: (i, 0)),
            pl.BlockSpec((RB, D_OUT), lambda i: (i, 0)),
        ],
        out_shape=[
            jax.ShapeDtypeStruct((N, D_OUT), jnp.float32),
            jax.ShapeDtypeStruct((N, D_OUT), jnp.float32),
        ],
    )(pa, pb, t2, dis, bmu, bls)


def kernel(x, edge_index, W1, b1, Wmu, bmu, Wls, bls):
    src = edge_index[0]
    dst = edge_index[1]
    ones16 = jnp.ones((NP, 16), jnp.float32)
    Wc = jnp.concatenate([Wmu, Wls], axis=1)
    dst3 = jnp.pad(dst.reshape(NW, EW), ((0, 0), (0, EWP - EW)),
                   constant_values=NP - 1).reshape(NW, NB, CH)

    dpA, dpB = _sc_deg(dst3, ones16)

    t1, dis = _tc_stage1(x, W1, dpA, dpB)

    s1a, s1b = _sc_prop(src, dst, t1)
    t2 = _tc_stage2(s1a, s1b, t1, dis, b1.reshape(1, -1), Wc)

    s2a, s2b = _sc_prop(src, dst, t2)
    mu, ls = _tc_stage3(s2a, s2b, t2, dis,
                        bmu.reshape(1, -1), bls.reshape(1, -1))
    return (mu, ls)
